# baseline (speedup 1.0000x reference)
import functools

import jax
import jax.numpy as jnp
from jax import lax
from jax.experimental import pallas as pl
from jax.experimental.pallas import tpu as pltpu
from jax.experimental.pallas import tpu_sc as plsc

N = 10000
NUM_NBR = 32
ATOM = 128
NBR_F = 16
E = N * NUM_NBR
F2 = 2 * ATOM
EPS = 1e-5

NW = 32
PER_W = E // NW
CH = 80
N_CH = PER_W // CH

G = 80
EB = G * NUM_NBR
NBLK = N // G


def _sigmoid(v):
    return 1.0 / (1.0 + jnp.exp(-v))


def _softplus(v):
    return jnp.maximum(v, 0.0) + jnp.log1p(jnp.exp(-jnp.abs(v)))


def _sc_gather(x, dst):
    width = x.shape[1]
    mesh = plsc.VectorSubcoreMesh(core_axis_name="c", subcore_axis_name="s")

    @functools.partial(
        pl.kernel,
        mesh=mesh,
        out_type=jax.ShapeDtypeStruct((E, width), x.dtype),
        scratch_types=[
            pltpu.VMEM((PER_W,), jnp.int32),
            pltpu.VMEM((2, CH, width), x.dtype),
            pltpu.SemaphoreType.DMA,
            pltpu.SemaphoreType.DMA,
        ],
    )
    def gk(x_hbm, dst_hbm, out_hbm, idx_v, rows_v, sem0, sem1):
        sems = (sem0, sem1)
        wid = lax.axis_index("s") * 2 + lax.axis_index("c")
        base = wid * PER_W
        pltpu.sync_copy(dst_hbm.at[pl.ds(base, PER_W)], idx_v)

        def gather_desc(c, bf):
            return pltpu.make_async_copy(
                x_hbm.at[idx_v.at[pl.ds(c * CH, CH)]], rows_v.at[bf], sems[bf])

        gather_desc(0, 0).start()
        gather_desc(1, 1).start()

        def body(g, carry):
            for bf in range(2):
                c = g * 2 + bf
                gather_desc(c, bf).wait()
                pltpu.sync_copy(rows_v.at[bf], out_hbm.at[pl.ds(base + c * CH, CH)])

                @pl.when(c + 2 < N_CH)
                def _():
                    gather_desc(c + 2, bf).start()

            return carry

        lax.fori_loop(0, N_CH // 2, body, 0)
        c_last = N_CH - 1
        gather_desc(c_last, c_last % 2).wait()
        pltpu.sync_copy(rows_v.at[c_last % 2],
                        out_hbm.at[pl.ds(base + c_last * CH, CH)])

    return gk(x, dst)


def _k_pre(x, wd):
    def body(x_r, wd_r, out_ref):
        out_ref[...] = jnp.dot(
            x_r[...].astype(jnp.bfloat16), wd_r[...],
            preferred_element_type=jnp.float32).astype(jnp.bfloat16)

    return pl.pallas_call(
        body,
        in_specs=[
            pl.BlockSpec((N, ATOM), lambda: (0, 0)),
            pl.BlockSpec((ATOM, F2), lambda: (0, 0)),
        ],
        out_specs=pl.BlockSpec((N, F2), lambda: (0, 0)),
        out_shape=jax.ShapeDtypeStruct((N, F2), jnp.bfloat16),
    )(x, wd)


def _k_stats(md, ea, x, we, ws, bb):
    def body(md_r, ea_r, x_r, we_r, ws_r, bb_r, s_ref, q_ref):
        eab = ea_r[...].astype(jnp.bfloat16)
        m = md_r[...].astype(jnp.float32)
        m = m + jnp.dot(eab, we_r[...], preferred_element_type=jnp.float32)
        a = jnp.dot(x_r[...], ws_r[...], preferred_element_type=jnp.float32)
        h = m.reshape(G, NUM_NBR, F2) + a[:, None, :] + bb_r[...][None]

        @pl.when(pl.program_id(0) == 0)
        def _():
            s_ref[...] = jnp.zeros_like(s_ref)
            q_ref[...] = jnp.zeros_like(q_ref)

        s_ref[...] += jnp.sum(h, axis=(0, 1))[None, :]
        q_ref[...] += jnp.sum(h * h, axis=(0, 1))[None, :]

    return pl.pallas_call(
        body,
        grid=(NBLK,),
        in_specs=[
            pl.BlockSpec((EB, F2), lambda i: (i, 0)),
            pl.BlockSpec((EB, NBR_F), lambda i: (i, 0)),
            pl.BlockSpec((G, ATOM), lambda i: (i, 0)),
            pl.BlockSpec((NBR_F, F2), lambda i: (0, 0)),
            pl.BlockSpec((ATOM, F2), lambda i: (0, 0)),
            pl.BlockSpec((1, F2), lambda i: (0, 0)),
        ],
        out_specs=[
            pl.BlockSpec((1, F2), lambda i: (0, 0)),
            pl.BlockSpec((1, F2), lambda i: (0, 0)),
        ],
        out_shape=[
            jax.ShapeDtypeStruct((1, F2), jnp.float32),
            jax.ShapeDtypeStruct((1, F2), jnp.float32),
        ],
    )(md, ea, x, we, ws, bb)


def _k_main(md, ea, x, we, ws, bb, s, q, g1, b1):
    def body(md_r, ea_r, x_r, we_r, ws_r, bb_r, s_r, q_r, g1_r, b1_r,
             out_ref, s2_ref, q2_ref):
        mu = s_r[...] * (1.0 / E)
        var = q_r[...] * (1.0 / E) - mu * mu
        inv = lax.rsqrt(var + EPS)
        scale = g1_r[...] * inv
        shift = b1_r[...] - mu * scale

        eab = ea_r[...].astype(jnp.bfloat16)
        m = md_r[...].astype(jnp.float32)
        m = m + jnp.dot(eab, we_r[...], preferred_element_type=jnp.float32)
        a = jnp.dot(x_r[...], ws_r[...], preferred_element_type=jnp.float32)
        h = m.reshape(G, NUM_NBR, F2) + a[:, None, :] + bb_r[...][None]
        hn = h * scale[None] + shift[None]
        filt = _sigmoid(hn[:, :, :ATOM])
        core = _softplus(hn[:, :, ATOM:])
        ssum = jnp.sum(filt * core, axis=1)
        out_ref[...] = ssum

        @pl.when(pl.program_id(0) == 0)
        def _():
            s2_ref[...] = jnp.zeros_like(s2_ref)
            q2_ref[...] = jnp.zeros_like(q2_ref)

        s2_ref[...] += jnp.sum(ssum, axis=0)[None, :]
        q2_ref[...] += jnp.sum(ssum * ssum, axis=0)[None, :]

    return pl.pallas_call(
        body,
        grid=(NBLK,),
        in_specs=[
            pl.BlockSpec((EB, F2), lambda i: (i, 0)),
            pl.BlockSpec((EB, NBR_F), lambda i: (i, 0)),
            pl.BlockSpec((G, ATOM), lambda i: (i, 0)),
            pl.BlockSpec((NBR_F, F2), lambda i: (0, 0)),
            pl.BlockSpec((ATOM, F2), lambda i: (0, 0)),
            pl.BlockSpec((1, F2), lambda i: (0, 0)),
            pl.BlockSpec((1, F2), lambda i: (0, 0)),
            pl.BlockSpec((1, F2), lambda i: (0, 0)),
            pl.BlockSpec((1, F2), lambda i: (0, 0)),
            pl.BlockSpec((1, F2), lambda i: (0, 0)),
        ],
        out_specs=[
            pl.BlockSpec((G, ATOM), lambda i: (i, 0)),
            pl.BlockSpec((1, ATOM), lambda i: (0, 0)),
            pl.BlockSpec((1, ATOM), lambda i: (0, 0)),
        ],
        out_shape=[
            jax.ShapeDtypeStruct((N, ATOM), jnp.float32),
            jax.ShapeDtypeStruct((1, ATOM), jnp.float32),
            jax.ShapeDtypeStruct((1, ATOM), jnp.float32),
        ],
    )(md, ea, x, we, ws, bb, s, q, g1, b1)


def _k_finish(x, ssum, s2, q2, g2, b2):
    def body(x_r, ss_r, s2_r, q2_r, g2_r, b2_r, out_ref):
        mu2 = s2_r[...] * (1.0 / N)
        var2 = q2_r[...] * (1.0 / N) - mu2 * mu2
        inv2 = lax.rsqrt(var2 + EPS)
        bn2 = (ss_r[...] - mu2) * (inv2 * g2_r[...]) + b2_r[...]
        out_ref[...] = _softplus(x_r[...] + bn2)

    return pl.pallas_call(
        body,
        grid=(NBLK,),
        in_specs=[
            pl.BlockSpec((G, ATOM), lambda i: (i, 0)),
            pl.BlockSpec((G, ATOM), lambda i: (i, 0)),
            pl.BlockSpec((1, ATOM), lambda i: (0, 0)),
            pl.BlockSpec((1, ATOM), lambda i: (0, 0)),
            pl.BlockSpec((1, ATOM), lambda i: (0, 0)),
            pl.BlockSpec((1, ATOM), lambda i: (0, 0)),
        ],
        out_specs=pl.BlockSpec((G, ATOM), lambda i: (i, 0)),
        out_shape=jax.ShapeDtypeStruct((N, ATOM), jnp.float32),
    )(x, ssum, s2, q2, g2, b2)


def kernel(x, edge_index, edge_attr, W, b, g1, b1, g2, b2):
    dst = edge_index[1].astype(jnp.int32)
    Wt = W.T
    wd = Wt[:ATOM].astype(jnp.bfloat16)
    ws = Wt[ATOM:2 * ATOM]
    we = Wt[2 * ATOM:].astype(jnp.bfloat16)
    bmat = _k_pre(x, wd)
    bb = b.reshape(1, F2)
    g1r = g1.reshape(1, F2)
    b1r = b1.reshape(1, F2)
    g2r = g2.reshape(1, ATOM)
    b2r = b2.reshape(1, ATOM)

    b_pk = lax.bitcast_convert_type(bmat.reshape(N, F2 // 2, 2), jnp.int32)
    md = lax.bitcast_convert_type(
        _sc_gather(b_pk, dst), jnp.bfloat16).reshape(E, F2)
    s, q = _k_stats(md, edge_attr, x, we, ws, bb)
    ssum, s2, q2 = _k_main(md, edge_attr, x, we, ws, bb, s, q, g1r, b1r)
    return _k_finish(x, ssum, s2, q2, g2r, b2r)


# speedup vs baseline: 2.8937x; 2.8937x over previous
import functools

import jax
import jax.numpy as jnp
from jax import lax
from jax.experimental import pallas as pl
from jax.experimental.pallas import tpu as pltpu
from jax.experimental.pallas import tpu_sc as plsc

N = 10000
NUM_NBR = 32
ATOM = 128
NBR_F = 16
E = N * NUM_NBR
F2 = 2 * ATOM
EPS = 1e-5

NW = 32
PER_W = E // NW
CH = 80
N_CH = PER_W // CH

G = 80
EB = G * NUM_NBR
NBLK = N // G

def _sigmoid(v):
    return 1.0 / (1.0 + jnp.exp(-v))


def _softplus(v):
    return jnp.maximum(v, 0.0) + jnp.log1p(jnp.exp(-jnp.abs(v)))


def _pack_halves(hi_f32, lo_f32):
    uh = lax.bitcast_convert_type(hi_f32, jnp.uint32)
    ul = lax.bitcast_convert_type(lo_f32, jnp.uint32)
    uh = uh + jnp.uint32(0x7FFF) + ((uh >> 16) & jnp.uint32(1))
    ul = ul + jnp.uint32(0x7FFF) + ((ul >> 16) & jnp.uint32(1))
    packed = (uh & jnp.uint32(0xFFFF0000)) | (ul >> 16)
    return lax.bitcast_convert_type(packed, jnp.int32)


def _unpack_halves(packed_i32):
    u = lax.bitcast_convert_type(packed_i32, jnp.uint32)
    hi = lax.bitcast_convert_type(u & jnp.uint32(0xFFFF0000), jnp.float32)
    lo = lax.bitcast_convert_type(u << 16, jnp.float32)
    return hi, lo


def _k_pre(x, wd):
    def body(x_r, wd_r, out_ref):
        mm = jnp.dot(x_r[...].astype(jnp.bfloat16), wd_r[...],
                     preferred_element_type=jnp.float32)
        out_ref[...] = _pack_halves(mm[:, :ATOM], mm[:, ATOM:])

    return pl.pallas_call(
        body,
        in_specs=[
            pl.BlockSpec((N, ATOM), lambda: (0, 0)),
            pl.BlockSpec((ATOM, F2), lambda: (0, 0)),
        ],
        out_specs=pl.BlockSpec((N, ATOM), lambda: (0, 0)),
        out_shape=jax.ShapeDtypeStruct((N, ATOM), jnp.int32),
    )(x, wd)


def _sc_gather(table, dst):
    mesh = plsc.VectorSubcoreMesh(core_axis_name="c", subcore_axis_name="s")

    @functools.partial(
        pl.kernel,
        mesh=mesh,
        out_type=jax.ShapeDtypeStruct((E, ATOM), jnp.int32),
        scratch_types=[
            pltpu.VMEM((PER_W,), jnp.int32),
            pltpu.VMEM((2, CH, ATOM), jnp.int32),
            pltpu.SemaphoreType.DMA,
            pltpu.SemaphoreType.DMA,
        ],
    )
    def gk(tab_hbm, dst_hbm, out_hbm, idx_v, rows_v, sem0, sem1):
        sems = (sem0, sem1)
        wid = lax.axis_index("s") * 2 + lax.axis_index("c")
        base = wid * PER_W
        pltpu.sync_copy(dst_hbm.at[pl.ds(base, PER_W)], idx_v)

        def gather_desc(c, bf):
            return pltpu.make_async_copy(
                tab_hbm.at[idx_v.at[pl.ds(c * CH, CH)]], rows_v.at[bf], sems[bf])

        gather_desc(0, 0).start()
        gather_desc(1, 1).start()

        def body(g, carry):
            for bf in range(2):
                c = g * 2 + bf
                gather_desc(c, bf).wait()
                pltpu.sync_copy(rows_v.at[bf], out_hbm.at[pl.ds(base + c * CH, CH)])

                @pl.when(c + 2 < N_CH)
                def _():
                    gather_desc(c + 2, bf).start()

            return carry

        lax.fori_loop(0, N_CH // 2, body, 0)
        c_last = N_CH - 1
        gather_desc(c_last, c_last % 2).wait()
        pltpu.sync_copy(rows_v.at[c_last % 2],
                        out_hbm.at[pl.ds(base + c_last * CH, CH)])

    return gk(table, dst)


def _edge_halves(md_r, ea_r, x_r, we_r, ws_r, bb_r):
    m_f, m_c = _unpack_halves(md_r[...])
    eab = ea_r[...].astype(jnp.bfloat16)
    ew = jnp.dot(eab, we_r[...], preferred_element_type=jnp.float32)
    a = jnp.dot(x_r[...], ws_r[...], preferred_element_type=jnp.float32)
    bb = bb_r[...]
    h_f = (m_f + ew[:, :ATOM]).reshape(G, NUM_NBR, ATOM) \
        + (a[:, :ATOM] + bb[:, :ATOM])[:, None, :]
    h_c = (m_c + ew[:, ATOM:]).reshape(G, NUM_NBR, ATOM) \
        + (a[:, ATOM:] + bb[:, ATOM:])[:, None, :]
    return h_f, h_c


def _k_stats(md, ea, x, we, ws, bb):
    def body(md_r, ea_r, x_r, we_r, ws_r, bb_r, s_ref, q_ref):
        h_f, h_c = _edge_halves(md_r, ea_r, x_r, we_r, ws_r, bb_r)

        @pl.when(pl.program_id(0) == 0)
        def _():
            s_ref[...] = jnp.zeros_like(s_ref)
            q_ref[...] = jnp.zeros_like(q_ref)

        s_ref[...] += jnp.concatenate(
            [jnp.sum(h_f, axis=(0, 1)), jnp.sum(h_c, axis=(0, 1))])[None, :]
        q_ref[...] += jnp.concatenate(
            [jnp.sum(h_f * h_f, axis=(0, 1)), jnp.sum(h_c * h_c, axis=(0, 1))])[None, :]

    return pl.pallas_call(
        body,
        grid=(NBLK,),
        in_specs=[
            pl.BlockSpec((EB, ATOM), lambda i: (i, 0)),
            pl.BlockSpec((EB, NBR_F), lambda i: (i, 0)),
            pl.BlockSpec((G, ATOM), lambda i: (i, 0)),
            pl.BlockSpec((NBR_F, F2), lambda i: (0, 0)),
            pl.BlockSpec((ATOM, F2), lambda i: (0, 0)),
            pl.BlockSpec((1, F2), lambda i: (0, 0)),
        ],
        out_specs=[
            pl.BlockSpec((1, F2), lambda i: (0, 0)),
            pl.BlockSpec((1, F2), lambda i: (0, 0)),
        ],
        out_shape=[
            jax.ShapeDtypeStruct((1, F2), jnp.float32),
            jax.ShapeDtypeStruct((1, F2), jnp.float32),
        ],
    )(md, ea, x, we, ws, bb)


def _k_main(md, ea, x, we, ws, bb, s, q, g1, b1):
    def body(md_r, ea_r, x_r, we_r, ws_r, bb_r, s_r, q_r, g1_r, b1_r,
             out_ref, s2_ref, q2_ref):
        mu = s_r[...] * (1.0 / E)
        var = q_r[...] * (1.0 / E) - mu * mu
        inv = lax.rsqrt(var + EPS)
        scale = g1_r[...] * inv
        shift = b1_r[...] - mu * scale

        h_f, h_c = _edge_halves(md_r, ea_r, x_r, we_r, ws_r, bb_r)
        hn_f = h_f * scale[:, None, :ATOM] + shift[:, None, :ATOM]
        hn_c = h_c * scale[:, None, ATOM:] + shift[:, None, ATOM:]
        filt = _sigmoid(hn_f)
        core = _softplus(hn_c)
        ssum = jnp.sum(filt * core, axis=1)
        out_ref[...] = ssum

        @pl.when(pl.program_id(0) == 0)
        def _():
            s2_ref[...] = jnp.zeros_like(s2_ref)
            q2_ref[...] = jnp.zeros_like(q2_ref)

        s2_ref[...] += jnp.sum(ssum, axis=0)[None, :]
        q2_ref[...] += jnp.sum(ssum * ssum, axis=0)[None, :]

    return pl.pallas_call(
        body,
        grid=(NBLK,),
        in_specs=[
            pl.BlockSpec((EB, ATOM), lambda i: (i, 0)),
            pl.BlockSpec((EB, NBR_F), lambda i: (i, 0)),
            pl.BlockSpec((G, ATOM), lambda i: (i, 0)),
            pl.BlockSpec((NBR_F, F2), lambda i: (0, 0)),
            pl.BlockSpec((ATOM, F2), lambda i: (0, 0)),
            pl.BlockSpec((1, F2), lambda i: (0, 0)),
            pl.BlockSpec((1, F2), lambda i: (0, 0)),
            pl.BlockSpec((1, F2), lambda i: (0, 0)),
            pl.BlockSpec((1, F2), lambda i: (0, 0)),
            pl.BlockSpec((1, F2), lambda i: (0, 0)),
        ],
        out_specs=[
            pl.BlockSpec((G, ATOM), lambda i: (i, 0)),
            pl.BlockSpec((1, ATOM), lambda i: (0, 0)),
            pl.BlockSpec((1, ATOM), lambda i: (0, 0)),
        ],
        out_shape=[
            jax.ShapeDtypeStruct((N, ATOM), jnp.float32),
            jax.ShapeDtypeStruct((1, ATOM), jnp.float32),
            jax.ShapeDtypeStruct((1, ATOM), jnp.float32),
        ],
    )(md, ea, x, we, ws, bb, s, q, g1, b1)


def _k_finish(x, ssum, s2, q2, g2, b2):
    def body(x_r, ss_r, s2_r, q2_r, g2_r, b2_r, out_ref):
        mu2 = s2_r[...] * (1.0 / N)
        var2 = q2_r[...] * (1.0 / N) - mu2 * mu2
        inv2 = lax.rsqrt(var2 + EPS)
        bn2 = (ss_r[...] - mu2) * (inv2 * g2_r[...]) + b2_r[...]
        out_ref[...] = _softplus(x_r[...] + bn2)

    return pl.pallas_call(
        body,
        grid=(NBLK,),
        in_specs=[
            pl.BlockSpec((G, ATOM), lambda i: (i, 0)),
            pl.BlockSpec((G, ATOM), lambda i: (i, 0)),
            pl.BlockSpec((1, ATOM), lambda i: (0, 0)),
            pl.BlockSpec((1, ATOM), lambda i: (0, 0)),
            pl.BlockSpec((1, ATOM), lambda i: (0, 0)),
            pl.BlockSpec((1, ATOM), lambda i: (0, 0)),
        ],
        out_specs=pl.BlockSpec((G, ATOM), lambda i: (i, 0)),
        out_shape=jax.ShapeDtypeStruct((N, ATOM), jnp.float32),
    )(x, ssum, s2, q2, g2, b2)


def kernel(x, edge_index, edge_attr, W, b, g1, b1, g2, b2):
    dst = edge_index[1].astype(jnp.int32)
    Wt = W.T
    wd = Wt[:ATOM].astype(jnp.bfloat16)
    ws = Wt[ATOM:2 * ATOM]
    we = Wt[2 * ATOM:].astype(jnp.bfloat16)
    bb = b.reshape(1, F2)
    g1r = g1.reshape(1, F2)
    b1r = b1.reshape(1, F2)
    g2r = g2.reshape(1, ATOM)
    b2r = b2.reshape(1, ATOM)

    bmat = _k_pre(x, wd)
    md = _sc_gather(bmat, dst)
    s, q = _k_stats(md, edge_attr, x, we, ws, bb)
    ssum, s2, q2 = _k_main(md, edge_attr, x, we, ws, bb, s, q, g1r, b1r)
    return _k_finish(x, ssum, s2, q2, g2r, b2r)


# speedup vs baseline: 3.5207x; 1.2167x over previous
import functools

import jax
import jax.numpy as jnp
from jax import lax
from jax.experimental import pallas as pl
from jax.experimental.pallas import tpu as pltpu
from jax.experimental.pallas import tpu_sc as plsc

N = 10000
NUM_NBR = 32
ATOM = 128
NBR_F = 16
E = N * NUM_NBR
F2 = 2 * ATOM
EPS = 1e-5

NW = 32
PER_W = E // NW
CH = 80
N_CH = PER_W // CH

G = 200
EB = G * NUM_NBR
NBLK = N // G

def _sigmoid(v):
    return 1.0 / (1.0 + jnp.exp(-v))


def _softplus(v):
    return jnp.maximum(v, 0.0) + jnp.log1p(jnp.exp(-jnp.abs(v)))


def _pack_halves(hi_f32, lo_f32):
    uh = lax.bitcast_convert_type(hi_f32, jnp.uint32)
    ul = lax.bitcast_convert_type(lo_f32, jnp.uint32)
    uh = uh + jnp.uint32(0x7FFF) + ((uh >> 16) & jnp.uint32(1))
    ul = ul + jnp.uint32(0x7FFF) + ((ul >> 16) & jnp.uint32(1))
    packed = (uh & jnp.uint32(0xFFFF0000)) | (ul >> 16)
    return lax.bitcast_convert_type(packed, jnp.int32)


def _unpack_halves(packed_i32):
    u = lax.bitcast_convert_type(packed_i32, jnp.uint32)
    hi = lax.bitcast_convert_type(u & jnp.uint32(0xFFFF0000), jnp.float32)
    lo = lax.bitcast_convert_type(u << 16, jnp.float32)
    return hi, lo


def _k_pre(x, wd):
    def body(x_r, wd_r, out_ref):
        mm = jnp.dot(x_r[...].astype(jnp.bfloat16), wd_r[...],
                     preferred_element_type=jnp.float32)
        out_ref[...] = _pack_halves(mm[:, :ATOM], mm[:, ATOM:])

    return pl.pallas_call(
        body,
        in_specs=[
            pl.BlockSpec((N, ATOM), lambda: (0, 0)),
            pl.BlockSpec((ATOM, F2), lambda: (0, 0)),
        ],
        out_specs=pl.BlockSpec((N, ATOM), lambda: (0, 0)),
        out_shape=jax.ShapeDtypeStruct((N, ATOM), jnp.int32),
    )(x, wd)


def _sc_gather(table, dst):
    mesh = plsc.VectorSubcoreMesh(core_axis_name="c", subcore_axis_name="s")

    @functools.partial(
        pl.kernel,
        mesh=mesh,
        out_type=jax.ShapeDtypeStruct((E, ATOM), jnp.int32),
        scratch_types=[
            pltpu.VMEM((PER_W,), jnp.int32),
            pltpu.VMEM((2, CH, ATOM), jnp.int32),
            pltpu.SemaphoreType.DMA,
            pltpu.SemaphoreType.DMA,
        ],
    )
    def gk(tab_hbm, dst_hbm, out_hbm, idx_v, rows_v, sem0, sem1):
        sems = (sem0, sem1)
        wid = lax.axis_index("s") * 2 + lax.axis_index("c")
        base = wid * PER_W
        pltpu.sync_copy(dst_hbm.at[pl.ds(base, PER_W)], idx_v)

        def gather_desc(c, bf):
            return pltpu.make_async_copy(
                tab_hbm.at[idx_v.at[pl.ds(c * CH, CH)]], rows_v.at[bf], sems[bf])

        gather_desc(0, 0).start()
        gather_desc(1, 1).start()

        def body(g, carry):
            for bf in range(2):
                c = g * 2 + bf
                gather_desc(c, bf).wait()
                pltpu.sync_copy(rows_v.at[bf], out_hbm.at[pl.ds(base + c * CH, CH)])

                @pl.when(c + 2 < N_CH)
                def _():
                    gather_desc(c + 2, bf).start()

            return carry

        lax.fori_loop(0, N_CH // 2, body, 0)
        c_last = N_CH - 1
        gather_desc(c_last, c_last % 2).wait()
        pltpu.sync_copy(rows_v.at[c_last % 2],
                        out_hbm.at[pl.ds(base + c_last * CH, CH)])

    return gk(table, dst)


def _edge_parts(md_r, ea_r, x_r, we_r, ws_r, bb_r):
    m_f, m_c = _unpack_halves(md_r[...])
    eab = ea_r[...].astype(jnp.bfloat16)
    ew = jnp.dot(eab, we_r[...], preferred_element_type=jnp.float32)
    a = jnp.dot(x_r[...], ws_r[...], preferred_element_type=jnp.float32)
    p_f = m_f + ew[:, :ATOM]
    p_c = m_c + ew[:, ATOM:]
    r = a + bb_r[...]
    return p_f, p_c, r


def _k_stats(md, ea, x, we, ws, bb):
    def body(md_r, ea_r, x_r, we_r, ws_r, bb_r, s_ref, q_ref):
        p_f, p_c, r = _edge_parts(md_r, ea_r, x_r, we_r, ws_r, bb_r)
        t_f = jnp.sum(p_f.reshape(G, NUM_NBR, ATOM), axis=1)
        t_c = jnp.sum(p_c.reshape(G, NUM_NBR, ATOM), axis=1)
        u_f = jnp.sum((p_f * p_f).reshape(G, NUM_NBR, ATOM), axis=1)
        u_c = jnp.sum((p_c * p_c).reshape(G, NUM_NBR, ATOM), axis=1)
        t = jnp.concatenate([t_f, t_c], axis=1)
        u = jnp.concatenate([u_f, u_c], axis=1)
        s_blk = jnp.sum(t + NUM_NBR * r, axis=0)
        q_blk = jnp.sum(u + 2.0 * (r * t) + NUM_NBR * (r * r), axis=0)

        @pl.when(pl.program_id(0) == 0)
        def _():
            s_ref[...] = jnp.zeros_like(s_ref)
            q_ref[...] = jnp.zeros_like(q_ref)

        s_ref[...] += s_blk[None, :]
        q_ref[...] += q_blk[None, :]

    return pl.pallas_call(
        body,
        grid=(NBLK,),
        in_specs=[
            pl.BlockSpec((EB, ATOM), lambda i: (i, 0)),
            pl.BlockSpec((EB, NBR_F), lambda i: (i, 0)),
            pl.BlockSpec((G, ATOM), lambda i: (i, 0)),
            pl.BlockSpec((NBR_F, F2), lambda i: (0, 0)),
            pl.BlockSpec((ATOM, F2), lambda i: (0, 0)),
            pl.BlockSpec((1, F2), lambda i: (0, 0)),
        ],
        out_specs=[
            pl.BlockSpec((1, F2), lambda i: (0, 0)),
            pl.BlockSpec((1, F2), lambda i: (0, 0)),
        ],
        out_shape=[
            jax.ShapeDtypeStruct((1, F2), jnp.float32),
            jax.ShapeDtypeStruct((1, F2), jnp.float32),
        ],
    )(md, ea, x, we, ws, bb)


def _k_main(md, ea, x, we, ws, bb, s, q, g1, b1):
    def body(md_r, ea_r, x_r, we_r, ws_r, bb_r, s_r, q_r, g1_r, b1_r,
             out_ref, s2_ref, q2_ref):
        mu = s_r[...] * (1.0 / E)
        var = q_r[...] * (1.0 / E) - mu * mu
        inv = lax.rsqrt(var + EPS)
        scale = g1_r[...] * inv
        shift = b1_r[...] - mu * scale

        p_f, p_c, r = _edge_parts(md_r, ea_r, x_r, we_r, ws_r, bb_r)
        shift_n = r * scale + shift
        hn_f = (p_f * scale[:, :ATOM]).reshape(G, NUM_NBR, ATOM) \
            + shift_n[:, None, :ATOM]
        hn_c = (p_c * scale[:, ATOM:]).reshape(G, NUM_NBR, ATOM) \
            + shift_n[:, None, ATOM:]
        filt = _sigmoid(hn_f)
        core = _softplus(hn_c)
        ssum = jnp.sum(filt * core, axis=1)
        out_ref[...] = ssum

        @pl.when(pl.program_id(0) == 0)
        def _():
            s2_ref[...] = jnp.zeros_like(s2_ref)
            q2_ref[...] = jnp.zeros_like(q2_ref)

        s2_ref[...] += jnp.sum(ssum, axis=0)[None, :]
        q2_ref[...] += jnp.sum(ssum * ssum, axis=0)[None, :]

    return pl.pallas_call(
        body,
        grid=(NBLK,),
        in_specs=[
            pl.BlockSpec((EB, ATOM), lambda i: (i, 0)),
            pl.BlockSpec((EB, NBR_F), lambda i: (i, 0)),
            pl.BlockSpec((G, ATOM), lambda i: (i, 0)),
            pl.BlockSpec((NBR_F, F2), lambda i: (0, 0)),
            pl.BlockSpec((ATOM, F2), lambda i: (0, 0)),
            pl.BlockSpec((1, F2), lambda i: (0, 0)),
            pl.BlockSpec((1, F2), lambda i: (0, 0)),
            pl.BlockSpec((1, F2), lambda i: (0, 0)),
            pl.BlockSpec((1, F2), lambda i: (0, 0)),
            pl.BlockSpec((1, F2), lambda i: (0, 0)),
        ],
        out_specs=[
            pl.BlockSpec((G, ATOM), lambda i: (i, 0)),
            pl.BlockSpec((1, ATOM), lambda i: (0, 0)),
            pl.BlockSpec((1, ATOM), lambda i: (0, 0)),
        ],
        out_shape=[
            jax.ShapeDtypeStruct((N, ATOM), jnp.float32),
            jax.ShapeDtypeStruct((1, ATOM), jnp.float32),
            jax.ShapeDtypeStruct((1, ATOM), jnp.float32),
        ],
    )(md, ea, x, we, ws, bb, s, q, g1, b1)


def _k_finish(x, ssum, s2, q2, g2, b2):
    def body(x_r, ss_r, s2_r, q2_r, g2_r, b2_r, out_ref):
        mu2 = s2_r[...] * (1.0 / N)
        var2 = q2_r[...] * (1.0 / N) - mu2 * mu2
        inv2 = lax.rsqrt(var2 + EPS)
        bn2 = (ss_r[...] - mu2) * (inv2 * g2_r[...]) + b2_r[...]
        out_ref[...] = _softplus(x_r[...] + bn2)

    return pl.pallas_call(
        body,
        grid=(NBLK,),
        in_specs=[
            pl.BlockSpec((G, ATOM), lambda i: (i, 0)),
            pl.BlockSpec((G, ATOM), lambda i: (i, 0)),
            pl.BlockSpec((1, ATOM), lambda i: (0, 0)),
            pl.BlockSpec((1, ATOM), lambda i: (0, 0)),
            pl.BlockSpec((1, ATOM), lambda i: (0, 0)),
            pl.BlockSpec((1, ATOM), lambda i: (0, 0)),
        ],
        out_specs=pl.BlockSpec((G, ATOM), lambda i: (i, 0)),
        out_shape=jax.ShapeDtypeStruct((N, ATOM), jnp.float32),
    )(x, ssum, s2, q2, g2, b2)


def kernel(x, edge_index, edge_attr, W, b, g1, b1, g2, b2):
    dst = edge_index[1].astype(jnp.int32)
    Wt = W.T
    wd = Wt[:ATOM].astype(jnp.bfloat16)
    ws = Wt[ATOM:2 * ATOM]
    we = Wt[2 * ATOM:].astype(jnp.bfloat16)
    bb = b.reshape(1, F2)
    g1r = g1.reshape(1, F2)
    b1r = b1.reshape(1, F2)
    g2r = g2.reshape(1, ATOM)
    b2r = b2.reshape(1, ATOM)

    bmat = _k_pre(x, wd)
    md = _sc_gather(bmat, dst)
    s, q = _k_stats(md, edge_attr, x, we, ws, bb)
    ssum, s2, q2 = _k_main(md, edge_attr, x, we, ws, bb, s, q, g1r, b1r)
    return _k_finish(x, ssum, s2, q2, g2r, b2r)


# speedup vs baseline: 3.5386x; 1.0051x over previous
import functools

import jax
import jax.numpy as jnp
from jax import lax
from jax.experimental import pallas as pl
from jax.experimental.pallas import tpu as pltpu
from jax.experimental.pallas import tpu_sc as plsc

N = 10000
NUM_NBR = 32
ATOM = 128
NBR_F = 16
E = N * NUM_NBR
F2 = 2 * ATOM
EPS = 1e-5

NW = 32
PER_W = E // NW
CH = 80
N_CH = PER_W // CH

G = 200
EB = G * NUM_NBR
NBLK = N // G

def _sigmoid(v):
    return 1.0 / (1.0 + jnp.exp(-v))


def _softplus(v):
    return jnp.maximum(v, 0.0) + jnp.log1p(jnp.exp(-jnp.abs(v)))


def _pack_halves(hi_f32, lo_f32):
    uh = lax.bitcast_convert_type(hi_f32, jnp.uint32)
    ul = lax.bitcast_convert_type(lo_f32, jnp.uint32)
    uh = uh + jnp.uint32(0x7FFF) + ((uh >> 16) & jnp.uint32(1))
    ul = ul + jnp.uint32(0x7FFF) + ((ul >> 16) & jnp.uint32(1))
    packed = (uh & jnp.uint32(0xFFFF0000)) | (ul >> 16)
    return lax.bitcast_convert_type(packed, jnp.int32)


def _unpack_halves(packed_i32):
    u = lax.bitcast_convert_type(packed_i32, jnp.uint32)
    hi = lax.bitcast_convert_type(u & jnp.uint32(0xFFFF0000), jnp.float32)
    lo = lax.bitcast_convert_type(u << 16, jnp.float32)
    return hi, lo


def _k_pre(x, wd):
    def body(x_r, wd_r, out_ref):
        mm = jnp.dot(x_r[...].astype(jnp.bfloat16), wd_r[...],
                     preferred_element_type=jnp.float32)
        out_ref[...] = _pack_halves(mm[:, :ATOM], mm[:, ATOM:])

    return pl.pallas_call(
        body,
        in_specs=[
            pl.BlockSpec((N, ATOM), lambda: (0, 0)),
            pl.BlockSpec((ATOM, F2), lambda: (0, 0)),
        ],
        out_specs=pl.BlockSpec((N, ATOM), lambda: (0, 0)),
        out_shape=jax.ShapeDtypeStruct((N, ATOM), jnp.int32),
    )(x, wd)


def _sc_gather(table, dst):
    mesh = plsc.VectorSubcoreMesh(core_axis_name="c", subcore_axis_name="s")

    NB = 8
    STAG = 4

    @functools.partial(
        pl.kernel,
        mesh=mesh,
        out_type=jax.ShapeDtypeStruct((E, ATOM), jnp.int32),
        scratch_types=[
            pltpu.VMEM((PER_W,), jnp.int32),
            pltpu.VMEM((NB, CH, ATOM), jnp.int32),
        ] + [pltpu.SemaphoreType.DMA] * (2 * NB),
    )
    def gk(tab_hbm, dst_hbm, out_hbm, idx_v, rows_v, *sems):
        gsems, ssems = sems[:NB], sems[NB:]
        wid = lax.axis_index("s") * 2 + lax.axis_index("c")
        base = wid * PER_W
        pltpu.sync_copy(dst_hbm.at[pl.ds(base, PER_W)], idx_v)

        def gdesc(c, bf):
            return pltpu.make_async_copy(
                tab_hbm.at[idx_v.at[pl.ds(c * CH, CH)]], rows_v.at[bf], gsems[bf])

        def sdesc(c, bf):
            return pltpu.make_async_copy(
                rows_v.at[bf], out_hbm.at[pl.ds(base + c * CH, CH)], ssems[bf])

        for c in range(STAG):
            gdesc(c, c % NB).start()

        def body(g, carry):
            for bf in range(NB):
                c = g * NB + bf

                @pl.when(c < N_CH)
                def _():
                    gdesc(c, bf).wait()
                    sdesc(c, bf).start()

                cn = c + STAG
                bfn = (bf + STAG) % NB

                @pl.when(cn < N_CH)
                def _():
                    @pl.when(cn >= NB)
                    def _():
                        sdesc(cn - NB, bfn).wait()

                    gdesc(cn, bfn).start()

            return carry

        lax.fori_loop(0, (N_CH + NB - 1) // NB, body, 0)
        for k in range(NB):
            c = N_CH - NB + k
            sdesc(c, c % NB).wait()

    return gk(table, dst)


def _edge_parts(md_r, ea_r, x_r, we_r, ws_r, bb_r):
    m_f, m_c = _unpack_halves(md_r[...])
    eab = ea_r[...].astype(jnp.bfloat16)
    ew = jnp.dot(eab, we_r[...], preferred_element_type=jnp.float32)
    a = jnp.dot(x_r[...], ws_r[...], preferred_element_type=jnp.float32)
    p_f = m_f + ew[:, :ATOM]
    p_c = m_c + ew[:, ATOM:]
    r = a + bb_r[...]
    return p_f, p_c, r


def _k_stats(md, ea, x, we, ws, bb):
    def body(md_r, ea_r, x_r, we_r, ws_r, bb_r, s_ref, q_ref):
        p_f, p_c, r = _edge_parts(md_r, ea_r, x_r, we_r, ws_r, bb_r)
        t_f = jnp.sum(p_f.reshape(G, NUM_NBR, ATOM), axis=1)
        t_c = jnp.sum(p_c.reshape(G, NUM_NBR, ATOM), axis=1)
        u_f = jnp.sum((p_f * p_f).reshape(G, NUM_NBR, ATOM), axis=1)
        u_c = jnp.sum((p_c * p_c).reshape(G, NUM_NBR, ATOM), axis=1)
        t = jnp.concatenate([t_f, t_c], axis=1)
        u = jnp.concatenate([u_f, u_c], axis=1)
        s_blk = jnp.sum(t + NUM_NBR * r, axis=0)
        q_blk = jnp.sum(u + 2.0 * (r * t) + NUM_NBR * (r * r), axis=0)

        @pl.when(pl.program_id(0) == 0)
        def _():
            s_ref[...] = jnp.zeros_like(s_ref)
            q_ref[...] = jnp.zeros_like(q_ref)

        s_ref[...] += s_blk[None, :]
        q_ref[...] += q_blk[None, :]

    return pl.pallas_call(
        body,
        grid=(NBLK,),
        in_specs=[
            pl.BlockSpec((EB, ATOM), lambda i: (i, 0)),
            pl.BlockSpec((EB, NBR_F), lambda i: (i, 0)),
            pl.BlockSpec((G, ATOM), lambda i: (i, 0)),
            pl.BlockSpec((NBR_F, F2), lambda i: (0, 0)),
            pl.BlockSpec((ATOM, F2), lambda i: (0, 0)),
            pl.BlockSpec((1, F2), lambda i: (0, 0)),
        ],
        out_specs=[
            pl.BlockSpec((1, F2), lambda i: (0, 0)),
            pl.BlockSpec((1, F2), lambda i: (0, 0)),
        ],
        out_shape=[
            jax.ShapeDtypeStruct((1, F2), jnp.float32),
            jax.ShapeDtypeStruct((1, F2), jnp.float32),
        ],
    )(md, ea, x, we, ws, bb)


def _k_main(md, ea, x, we, ws, bb, s, q, g1, b1):
    def body(md_r, ea_r, x_r, we_r, ws_r, bb_r, s_r, q_r, g1_r, b1_r,
             out_ref, s2_ref, q2_ref):
        mu = s_r[...] * (1.0 / E)
        var = q_r[...] * (1.0 / E) - mu * mu
        inv = lax.rsqrt(var + EPS)
        scale = g1_r[...] * inv
        shift = b1_r[...] - mu * scale

        p_f, p_c, r = _edge_parts(md_r, ea_r, x_r, we_r, ws_r, bb_r)
        shift_n = r * scale + shift
        hn_f = (p_f * scale[:, :ATOM]).reshape(G, NUM_NBR, ATOM) \
            + shift_n[:, None, :ATOM]
        hn_c = (p_c * scale[:, ATOM:]).reshape(G, NUM_NBR, ATOM) \
            + shift_n[:, None, ATOM:]
        filt = _sigmoid(hn_f)
        core = _softplus(hn_c)
        ssum = jnp.sum(filt * core, axis=1)
        out_ref[...] = ssum

        @pl.when(pl.program_id(0) == 0)
        def _():
            s2_ref[...] = jnp.zeros_like(s2_ref)
            q2_ref[...] = jnp.zeros_like(q2_ref)

        s2_ref[...] += jnp.sum(ssum, axis=0)[None, :]
        q2_ref[...] += jnp.sum(ssum * ssum, axis=0)[None, :]

    return pl.pallas_call(
        body,
        grid=(NBLK,),
        in_specs=[
            pl.BlockSpec((EB, ATOM), lambda i: (i, 0)),
            pl.BlockSpec((EB, NBR_F), lambda i: (i, 0)),
            pl.BlockSpec((G, ATOM), lambda i: (i, 0)),
            pl.BlockSpec((NBR_F, F2), lambda i: (0, 0)),
            pl.BlockSpec((ATOM, F2), lambda i: (0, 0)),
            pl.BlockSpec((1, F2), lambda i: (0, 0)),
            pl.BlockSpec((1, F2), lambda i: (0, 0)),
            pl.BlockSpec((1, F2), lambda i: (0, 0)),
            pl.BlockSpec((1, F2), lambda i: (0, 0)),
            pl.BlockSpec((1, F2), lambda i: (0, 0)),
        ],
        out_specs=[
            pl.BlockSpec((G, ATOM), lambda i: (i, 0)),
            pl.BlockSpec((1, ATOM), lambda i: (0, 0)),
            pl.BlockSpec((1, ATOM), lambda i: (0, 0)),
        ],
        out_shape=[
            jax.ShapeDtypeStruct((N, ATOM), jnp.float32),
            jax.ShapeDtypeStruct((1, ATOM), jnp.float32),
            jax.ShapeDtypeStruct((1, ATOM), jnp.float32),
        ],
    )(md, ea, x, we, ws, bb, s, q, g1, b1)


def _k_finish(x, ssum, s2, q2, g2, b2):
    def body(x_r, ss_r, s2_r, q2_r, g2_r, b2_r, out_ref):
        mu2 = s2_r[...] * (1.0 / N)
        var2 = q2_r[...] * (1.0 / N) - mu2 * mu2
        inv2 = lax.rsqrt(var2 + EPS)
        bn2 = (ss_r[...] - mu2) * (inv2 * g2_r[...]) + b2_r[...]
        out_ref[...] = _softplus(x_r[...] + bn2)

    return pl.pallas_call(
        body,
        grid=(NBLK,),
        in_specs=[
            pl.BlockSpec((G, ATOM), lambda i: (i, 0)),
            pl.BlockSpec((G, ATOM), lambda i: (i, 0)),
            pl.BlockSpec((1, ATOM), lambda i: (0, 0)),
            pl.BlockSpec((1, ATOM), lambda i: (0, 0)),
            pl.BlockSpec((1, ATOM), lambda i: (0, 0)),
            pl.BlockSpec((1, ATOM), lambda i: (0, 0)),
        ],
        out_specs=pl.BlockSpec((G, ATOM), lambda i: (i, 0)),
        out_shape=jax.ShapeDtypeStruct((N, ATOM), jnp.float32),
    )(x, ssum, s2, q2, g2, b2)


def kernel(x, edge_index, edge_attr, W, b, g1, b1, g2, b2):
    dst = edge_index[1].astype(jnp.int32)
    Wt = W.T
    wd = Wt[:ATOM].astype(jnp.bfloat16)
    ws = Wt[ATOM:2 * ATOM]
    we = Wt[2 * ATOM:].astype(jnp.bfloat16)
    bb = b.reshape(1, F2)
    g1r = g1.reshape(1, F2)
    b1r = b1.reshape(1, F2)
    g2r = g2.reshape(1, ATOM)
    b2r = b2.reshape(1, ATOM)

    bmat = _k_pre(x, wd)
    md = _sc_gather(bmat, dst)
    s, q = _k_stats(md, edge_attr, x, we, ws, bb)
    ssum, s2, q2 = _k_main(md, edge_attr, x, we, ws, bb, s, q, g1r, b1r)
    return _k_finish(x, ssum, s2, q2, g2r, b2r)


# speedup vs baseline: 3.5924x; 1.0152x over previous
import functools

import jax
import jax.numpy as jnp
from jax import lax
from jax.experimental import pallas as pl
from jax.experimental.pallas import tpu as pltpu
from jax.experimental.pallas import tpu_sc as plsc

N = 10000
NUM_NBR = 32
ATOM = 128
NBR_F = 16
E = N * NUM_NBR
F2 = 2 * ATOM
EPS = 1e-5

NW = 32
PER_W = E // NW
CH = 80
N_CH = PER_W // CH

G = 200
EB = G * NUM_NBR
NBLK = N // G

def _sigmoid(v):
    return 1.0 / (1.0 + jnp.exp(-v))


def _softplus(v):
    return jnp.maximum(v, 0.0) + jnp.log1p(jnp.exp(-jnp.abs(v)))


def _pack_halves(hi_f32, lo_f32):
    uh = lax.bitcast_convert_type(hi_f32, jnp.uint32)
    ul = lax.bitcast_convert_type(lo_f32, jnp.uint32)
    uh = uh + jnp.uint32(0x7FFF) + ((uh >> 16) & jnp.uint32(1))
    ul = ul + jnp.uint32(0x7FFF) + ((ul >> 16) & jnp.uint32(1))
    packed = (uh & jnp.uint32(0xFFFF0000)) | (ul >> 16)
    return lax.bitcast_convert_type(packed, jnp.int32)


def _unpack_halves(packed_i32):
    u = lax.bitcast_convert_type(packed_i32, jnp.uint32)
    hi = lax.bitcast_convert_type(u & jnp.uint32(0xFFFF0000), jnp.float32)
    lo = lax.bitcast_convert_type(u << 16, jnp.float32)
    return hi, lo


def _k_pre(x, wd):
    def body(x_r, wd_r, out_ref):
        mm = jnp.dot(x_r[...].astype(jnp.bfloat16), wd_r[...],
                     preferred_element_type=jnp.float32)
        out_ref[...] = _pack_halves(mm[:, :ATOM], mm[:, ATOM:])

    return pl.pallas_call(
        body,
        in_specs=[
            pl.BlockSpec((N, ATOM), lambda: (0, 0)),
            pl.BlockSpec((ATOM, F2), lambda: (0, 0)),
        ],
        out_specs=pl.BlockSpec((N, ATOM), lambda: (0, 0)),
        out_shape=jax.ShapeDtypeStruct((N, ATOM), jnp.int32),
    )(x, wd)


def _sc_gather(table, dst):
    mesh = plsc.VectorSubcoreMesh(core_axis_name="c", subcore_axis_name="s")

    NB = 8
    STAG = 4

    @functools.partial(
        pl.kernel,
        mesh=mesh,
        out_type=jax.ShapeDtypeStruct((E, ATOM), jnp.int32),
        scratch_types=[
            pltpu.VMEM((PER_W,), jnp.int32),
            pltpu.VMEM((NB, CH, ATOM), jnp.int32),
        ] + [pltpu.SemaphoreType.DMA] * (2 * NB),
    )
    def gk(tab_hbm, dst_hbm, out_hbm, idx_v, rows_v, *sems):
        gsems, ssems = sems[:NB], sems[NB:]
        wid = lax.axis_index("s") * 2 + lax.axis_index("c")
        base = wid * PER_W
        pltpu.sync_copy(dst_hbm.at[pl.ds(base, PER_W)], idx_v)

        def gdesc(c, bf):
            return pltpu.make_async_copy(
                tab_hbm.at[idx_v.at[pl.ds(c * CH, CH)]], rows_v.at[bf], gsems[bf])

        def sdesc(c, bf):
            return pltpu.make_async_copy(
                rows_v.at[bf], out_hbm.at[pl.ds(base + c * CH, CH)], ssems[bf])

        for c in range(STAG):
            gdesc(c, c % NB).start()

        def body(g, carry):
            for bf in range(NB):
                c = g * NB + bf

                @pl.when(c < N_CH)
                def _():
                    gdesc(c, bf).wait()
                    sdesc(c, bf).start()

                cn = c + STAG
                bfn = (bf + STAG) % NB

                @pl.when(cn < N_CH)
                def _():
                    @pl.when(cn >= NB)
                    def _():
                        sdesc(cn - NB, bfn).wait()

                    gdesc(cn, bfn).start()

            return carry

        lax.fori_loop(0, (N_CH + NB - 1) // NB, body, 0)
        for k in range(NB):
            c = N_CH - NB + k
            sdesc(c, c % NB).wait()

    return gk(table, dst)


def _edge_parts(md_r, ea_r, x_r, we_r, ws_r, bb_r):
    m_f, m_c = _unpack_halves(md_r[...])
    eab = ea_r[...].astype(jnp.bfloat16)
    ew = jnp.dot(eab, we_r[...], preferred_element_type=jnp.float32)
    a = jnp.dot(x_r[...], ws_r[...], preferred_element_type=jnp.float32)
    p_f = m_f + ew[:, :ATOM]
    p_c = m_c + ew[:, ATOM:]
    r = a + bb_r[...]
    return p_f, p_c, r


PH = NBLK


def _k_fused(md, ea, x, we, ws, bb, g1, b1, g2, b2):
    def body(md_r, ea_r, x_r, we_r, ws_r, bb_r, g1_r, b1_r, g2_r, b2_r,
             out_ref, sq_v, s2q2_v, s_v):
        i = pl.program_id(0)

        @pl.when(i == 0)
        def _():
            sq_v[...] = jnp.zeros_like(sq_v)
            s2q2_v[...] = jnp.zeros_like(s2q2_v)

        @pl.when(i < PH)
        def _():
            p_f, p_c, r = _edge_parts(md_r, ea_r, x_r, we_r, ws_r, bb_r)
            t_f = jnp.sum(p_f.reshape(G, NUM_NBR, ATOM), axis=1)
            t_c = jnp.sum(p_c.reshape(G, NUM_NBR, ATOM), axis=1)
            u_f = jnp.sum((p_f * p_f).reshape(G, NUM_NBR, ATOM), axis=1)
            u_c = jnp.sum((p_c * p_c).reshape(G, NUM_NBR, ATOM), axis=1)
            t = jnp.concatenate([t_f, t_c], axis=1)
            u = jnp.concatenate([u_f, u_c], axis=1)
            s_blk = jnp.sum(t + NUM_NBR * r, axis=0)
            q_blk = jnp.sum(u + 2.0 * (r * t) + NUM_NBR * (r * r), axis=0)
            sq_v[0:1, :] += s_blk[None, :]
            sq_v[1:2, :] += q_blk[None, :]

        @pl.when((i >= PH) & (i < 2 * PH))
        def _():
            mu = sq_v[0:1, :] * (1.0 / E)
            var = sq_v[1:2, :] * (1.0 / E) - mu * mu
            inv = lax.rsqrt(var + EPS)
            scale = g1_r[...] * inv
            shift = b1_r[...] - mu * scale

            p_f, p_c, r = _edge_parts(md_r, ea_r, x_r, we_r, ws_r, bb_r)
            shift_n = r * scale + shift
            hn_f = (p_f * scale[:, :ATOM]).reshape(G, NUM_NBR, ATOM) \
                + shift_n[:, None, :ATOM]
            hn_c = (p_c * scale[:, ATOM:]).reshape(G, NUM_NBR, ATOM) \
                + shift_n[:, None, ATOM:]
            ssum = jnp.sum(_sigmoid(hn_f) * _softplus(hn_c), axis=1)
            j = i - PH
            s_v[pl.ds(j * G, G), :] = ssum
            s2q2_v[0:1, :] += jnp.sum(ssum, axis=0)[None, :]
            s2q2_v[1:2, :] += jnp.sum(ssum * ssum, axis=0)[None, :]

        @pl.when(i >= 2 * PH)
        def _():
            mu2 = s2q2_v[0:1, :] * (1.0 / N)
            var2 = s2q2_v[1:2, :] * (1.0 / N) - mu2 * mu2
            inv2 = lax.rsqrt(var2 + EPS)
            j = i - 2 * PH
            ss = s_v[pl.ds(j * G, G), :]
            bn2 = (ss - mu2) * (inv2 * g2_r[...]) + b2_r[...]
            out_ref[...] = _softplus(x_r[...] + bn2)

    def _md_map(i):
        return (jnp.where(i < PH, i, jnp.minimum(i - PH, PH - 1)), 0)

    def _x_map(i):
        return (lax.rem(i, PH), 0)

    def _out_map(i):
        return (jnp.maximum(i - 2 * PH, 0), 0)

    return pl.pallas_call(
        body,
        grid=(3 * PH,),
        in_specs=[
            pl.BlockSpec((EB, ATOM), _md_map),
            pl.BlockSpec((EB, NBR_F), _md_map),
            pl.BlockSpec((G, ATOM), _x_map),
            pl.BlockSpec((NBR_F, F2), lambda i: (0, 0)),
            pl.BlockSpec((ATOM, F2), lambda i: (0, 0)),
            pl.BlockSpec((1, F2), lambda i: (0, 0)),
            pl.BlockSpec((1, F2), lambda i: (0, 0)),
            pl.BlockSpec((1, F2), lambda i: (0, 0)),
            pl.BlockSpec((1, ATOM), lambda i: (0, 0)),
            pl.BlockSpec((1, ATOM), lambda i: (0, 0)),
        ],
        out_specs=pl.BlockSpec((G, ATOM), _out_map),
        out_shape=jax.ShapeDtypeStruct((N, ATOM), jnp.float32),
        scratch_shapes=[
            pltpu.VMEM((2, F2), jnp.float32),
            pltpu.VMEM((2, ATOM), jnp.float32),
            pltpu.VMEM((N, ATOM), jnp.float32),
        ],
    )(md, ea, x, we, ws, bb, g1, b1, g2, b2)


def kernel(x, edge_index, edge_attr, W, b, g1, b1, g2, b2):
    dst = edge_index[1].astype(jnp.int32)
    Wt = W.T
    wd = Wt[:ATOM].astype(jnp.bfloat16)
    ws = Wt[ATOM:2 * ATOM]
    we = Wt[2 * ATOM:].astype(jnp.bfloat16)
    bb = b.reshape(1, F2)
    g1r = g1.reshape(1, F2)
    b1r = b1.reshape(1, F2)
    g2r = g2.reshape(1, ATOM)
    b2r = b2.reshape(1, ATOM)

    bmat = _k_pre(x, wd)
    md = _sc_gather(bmat, dst)
    return _k_fused(md, edge_attr, x, we, ws, bb, g1r, b1r, g2r, b2r)


# speedup vs baseline: 3.9141x; 1.0895x over previous
import functools

import jax
import jax.numpy as jnp
from jax import lax
from jax.experimental import pallas as pl
from jax.experimental.pallas import tpu as pltpu
from jax.experimental.pallas import tpu_sc as plsc

N = 10000
NUM_NBR = 32
ATOM = 128
NBR_F = 16
E = N * NUM_NBR
F2 = 2 * ATOM
EPS = 1e-5

NW = 32
PER_W = E // NW
CH = 80
N_CH = PER_W // CH

G = 200
EB = G * NUM_NBR
NBLK = N // G

_LN2 = 0.6931471805599453
_LOG2E = 1.4426950408889634


def _sigmoid(v):
    return 0.5 * jnp.tanh(0.5 * v) + 0.5


def _softplus(v):
    return jnp.maximum(v, 0.0) + _LN2 * jnp.log2(1.0 + jnp.exp2(-_LOG2E * jnp.abs(v)))


def _pack_halves(hi_f32, lo_f32):
    uh = lax.bitcast_convert_type(hi_f32, jnp.uint32)
    ul = lax.bitcast_convert_type(lo_f32, jnp.uint32)
    uh = uh + jnp.uint32(0x7FFF) + ((uh >> 16) & jnp.uint32(1))
    ul = ul + jnp.uint32(0x7FFF) + ((ul >> 16) & jnp.uint32(1))
    packed = (uh & jnp.uint32(0xFFFF0000)) | (ul >> 16)
    return lax.bitcast_convert_type(packed, jnp.int32)


def _unpack_halves(packed_i32):
    u = lax.bitcast_convert_type(packed_i32, jnp.uint32)
    hi = lax.bitcast_convert_type(u & jnp.uint32(0xFFFF0000), jnp.float32)
    lo = lax.bitcast_convert_type(u << 16, jnp.float32)
    return hi, lo


def _k_pre(x, wd):
    def body(x_r, wd_r, out_ref):
        mm = jnp.dot(x_r[...].astype(jnp.bfloat16), wd_r[...],
                     preferred_element_type=jnp.float32)
        out_ref[...] = _pack_halves(mm[:, :ATOM], mm[:, ATOM:])

    return pl.pallas_call(
        body,
        in_specs=[
            pl.BlockSpec((N, ATOM), lambda: (0, 0)),
            pl.BlockSpec((ATOM, F2), lambda: (0, 0)),
        ],
        out_specs=pl.BlockSpec((N, ATOM), lambda: (0, 0)),
        out_shape=jax.ShapeDtypeStruct((N, ATOM), jnp.int32),
    )(x, wd)


def _sc_gather(table, dst):
    mesh = plsc.VectorSubcoreMesh(core_axis_name="c", subcore_axis_name="s")

    NB = 8
    STAG = 4

    @functools.partial(
        pl.kernel,
        mesh=mesh,
        out_type=jax.ShapeDtypeStruct((E, ATOM), jnp.int32),
        scratch_types=[
            pltpu.VMEM((PER_W,), jnp.int32),
            pltpu.VMEM((NB, CH, ATOM), jnp.int32),
        ] + [pltpu.SemaphoreType.DMA] * (2 * NB),
    )
    def gk(tab_hbm, dst_hbm, out_hbm, idx_v, rows_v, *sems):
        gsems, ssems = sems[:NB], sems[NB:]
        wid = lax.axis_index("s") * 2 + lax.axis_index("c")
        base = wid * PER_W
        pltpu.sync_copy(dst_hbm.at[pl.ds(base, PER_W)], idx_v)

        def gdesc(c, bf):
            return pltpu.make_async_copy(
                tab_hbm.at[idx_v.at[pl.ds(c * CH, CH)]], rows_v.at[bf], gsems[bf])

        def sdesc(c, bf):
            return pltpu.make_async_copy(
                rows_v.at[bf], out_hbm.at[pl.ds(base + c * CH, CH)], ssems[bf])

        for c in range(STAG):
            gdesc(c, c % NB).start()

        def body(g, carry):
            for bf in range(NB):
                c = g * NB + bf

                @pl.when(c < N_CH)
                def _():
                    gdesc(c, bf).wait()
                    sdesc(c, bf).start()

                cn = c + STAG
                bfn = (bf + STAG) % NB

                @pl.when(cn < N_CH)
                def _():
                    @pl.when(cn >= NB)
                    def _():
                        sdesc(cn - NB, bfn).wait()

                    gdesc(cn, bfn).start()

            return carry

        lax.fori_loop(0, (N_CH + NB - 1) // NB, body, 0)
        for k in range(NB):
            c = N_CH - NB + k
            sdesc(c, c % NB).wait()

    return gk(table, dst)


def _edge_parts(md_r, ea_r, x_r, we_r, ws_r, bb_r):
    m_f, m_c = _unpack_halves(md_r[...])
    eab = ea_r[...].astype(jnp.bfloat16)
    ew = jnp.dot(eab, we_r[...], preferred_element_type=jnp.float32)
    a = jnp.dot(x_r[...], ws_r[...], preferred_element_type=jnp.float32)
    p_f = m_f + ew[:, :ATOM]
    p_c = m_c + ew[:, ATOM:]
    r = a + bb_r[...]
    return p_f, p_c, r


PH = NBLK


def _k_fused(md, ea, x, we, ws, bb, g1, b1, g2, b2):
    def body(md_r, ea_r, x_r, we_r, ws_r, bb_r, g1_r, b1_r, g2_r, b2_r,
             out_ref, sq_v, s2q2_v, s_v):
        i = pl.program_id(0)

        @pl.when(i == 0)
        def _():
            sq_v[...] = jnp.zeros_like(sq_v)
            s2q2_v[...] = jnp.zeros_like(s2q2_v)

        @pl.when(i < PH)
        def _():
            p_f, p_c, r = _edge_parts(md_r, ea_r, x_r, we_r, ws_r, bb_r)
            t_f = jnp.sum(p_f.reshape(G, NUM_NBR, ATOM), axis=1)
            t_c = jnp.sum(p_c.reshape(G, NUM_NBR, ATOM), axis=1)
            u_f = jnp.sum((p_f * p_f).reshape(G, NUM_NBR, ATOM), axis=1)
            u_c = jnp.sum((p_c * p_c).reshape(G, NUM_NBR, ATOM), axis=1)
            t = jnp.concatenate([t_f, t_c], axis=1)
            u = jnp.concatenate([u_f, u_c], axis=1)
            s_blk = jnp.sum(t + NUM_NBR * r, axis=0)
            q_blk = jnp.sum(u + 2.0 * (r * t) + NUM_NBR * (r * r), axis=0)
            sq_v[0:1, :] += s_blk[None, :]
            sq_v[1:2, :] += q_blk[None, :]

        @pl.when((i >= PH) & (i < 2 * PH))
        def _():
            mu = sq_v[0:1, :] * (1.0 / E)
            var = sq_v[1:2, :] * (1.0 / E) - mu * mu
            inv = lax.rsqrt(var + EPS)
            scale = g1_r[...] * inv
            shift = b1_r[...] - mu * scale

            p_f, p_c, r = _edge_parts(md_r, ea_r, x_r, we_r, ws_r, bb_r)
            shift_n = r * scale + shift
            hn_fh = (p_f * (0.5 * scale[:, :ATOM])).reshape(G, NUM_NBR, ATOM) \
                + (0.5 * shift_n[:, None, :ATOM])
            hn_c = (p_c * scale[:, ATOM:]).reshape(G, NUM_NBR, ATOM) \
                + shift_n[:, None, ATOM:]
            gate = (jnp.tanh(hn_fh) + 1.0) * _softplus(hn_c)
            ssum = 0.5 * jnp.sum(gate, axis=1)
            j = i - PH
            s_v[pl.ds(j * G, G), :] = ssum
            s2q2_v[0:1, :] += jnp.sum(ssum, axis=0)[None, :]
            s2q2_v[1:2, :] += jnp.sum(ssum * ssum, axis=0)[None, :]

        @pl.when(i >= 2 * PH)
        def _():
            mu2 = s2q2_v[0:1, :] * (1.0 / N)
            var2 = s2q2_v[1:2, :] * (1.0 / N) - mu2 * mu2
            inv2 = lax.rsqrt(var2 + EPS)
            j = i - 2 * PH
            ss = s_v[pl.ds(j * G, G), :]
            bn2 = (ss - mu2) * (inv2 * g2_r[...]) + b2_r[...]
            out_ref[...] = _softplus(x_r[...] + bn2)

    def _md_map(i):
        return (jnp.where(i < PH, i, jnp.minimum(i - PH, PH - 1)), 0)

    def _x_map(i):
        return (lax.rem(i, PH), 0)

    def _out_map(i):
        return (jnp.maximum(i - 2 * PH, 0), 0)

    return pl.pallas_call(
        body,
        grid=(3 * PH,),
        in_specs=[
            pl.BlockSpec((EB, ATOM), _md_map),
            pl.BlockSpec((EB, NBR_F), _md_map),
            pl.BlockSpec((G, ATOM), _x_map),
            pl.BlockSpec((NBR_F, F2), lambda i: (0, 0)),
            pl.BlockSpec((ATOM, F2), lambda i: (0, 0)),
            pl.BlockSpec((1, F2), lambda i: (0, 0)),
            pl.BlockSpec((1, F2), lambda i: (0, 0)),
            pl.BlockSpec((1, F2), lambda i: (0, 0)),
            pl.BlockSpec((1, ATOM), lambda i: (0, 0)),
            pl.BlockSpec((1, ATOM), lambda i: (0, 0)),
        ],
        out_specs=pl.BlockSpec((G, ATOM), _out_map),
        out_shape=jax.ShapeDtypeStruct((N, ATOM), jnp.float32),
        scratch_shapes=[
            pltpu.VMEM((2, F2), jnp.float32),
            pltpu.VMEM((2, ATOM), jnp.float32),
            pltpu.VMEM((N, ATOM), jnp.float32),
        ],
    )(md, ea, x, we, ws, bb, g1, b1, g2, b2)


def kernel(x, edge_index, edge_attr, W, b, g1, b1, g2, b2):
    dst = edge_index[1].astype(jnp.int32)
    Wt = W.T
    wd = Wt[:ATOM].astype(jnp.bfloat16)
    ws = Wt[ATOM:2 * ATOM]
    we = Wt[2 * ATOM:].astype(jnp.bfloat16)
    bb = b.reshape(1, F2)
    g1r = g1.reshape(1, F2)
    b1r = b1.reshape(1, F2)
    g2r = g2.reshape(1, ATOM)
    b2r = b2.reshape(1, ATOM)

    bmat = _k_pre(x, wd)
    md = _sc_gather(bmat, dst)
    return _k_fused(md, edge_attr, x, we, ws, bb, g1r, b1r, g2r, b2r)


# speedup vs baseline: 4.1484x; 1.0599x over previous
import functools

import jax
import jax.numpy as jnp
from jax import lax
from jax.experimental import pallas as pl
from jax.experimental.pallas import tpu as pltpu
from jax.experimental.pallas import tpu_sc as plsc

N = 10000
NUM_NBR = 32
ATOM = 128
NBR_F = 16
E = N * NUM_NBR
F2 = 2 * ATOM
EPS = 1e-5

NW = 32
PER_W = E // NW
CH = 80
N_CH = PER_W // CH

G = 400
EB = G * NUM_NBR
NBLK = N // G

_LN2 = 0.6931471805599453
_LOG2E = 1.4426950408889634


def _sigmoid(v):
    return 0.5 * jnp.tanh(0.5 * v) + 0.5


def _softplus(v):
    return jnp.maximum(v, 0.0) + _LN2 * jnp.log2(1.0 + jnp.exp2(-_LOG2E * jnp.abs(v)))


def _pack_halves(hi_f32, lo_f32):
    uh = lax.bitcast_convert_type(hi_f32, jnp.uint32)
    ul = lax.bitcast_convert_type(lo_f32, jnp.uint32)
    uh = uh + jnp.uint32(0x7FFF) + ((uh >> 16) & jnp.uint32(1))
    ul = ul + jnp.uint32(0x7FFF) + ((ul >> 16) & jnp.uint32(1))
    packed = (uh & jnp.uint32(0xFFFF0000)) | (ul >> 16)
    return lax.bitcast_convert_type(packed, jnp.int32)


def _unpack_halves(packed_i32):
    u = lax.bitcast_convert_type(packed_i32, jnp.uint32)
    hi = lax.bitcast_convert_type(u & jnp.uint32(0xFFFF0000), jnp.float32)
    lo = lax.bitcast_convert_type(u << 16, jnp.float32)
    return hi, lo


def _k_pre(x, wd):
    def body(x_r, wd_r, out_ref):
        mm = jnp.dot(x_r[...].astype(jnp.bfloat16), wd_r[...],
                     preferred_element_type=jnp.float32)
        out_ref[...] = _pack_halves(mm[:, :ATOM], mm[:, ATOM:])

    return pl.pallas_call(
        body,
        in_specs=[
            pl.BlockSpec((N, ATOM), lambda: (0, 0)),
            pl.BlockSpec((ATOM, F2), lambda: (0, 0)),
        ],
        out_specs=pl.BlockSpec((N, ATOM), lambda: (0, 0)),
        out_shape=jax.ShapeDtypeStruct((N, ATOM), jnp.int32),
    )(x, wd)


def _sc_gather(table, dst):
    mesh = plsc.VectorSubcoreMesh(core_axis_name="c", subcore_axis_name="s")

    NB = 8
    STAG = 4

    @functools.partial(
        pl.kernel,
        mesh=mesh,
        out_type=jax.ShapeDtypeStruct((E, ATOM), jnp.int32),
        scratch_types=[
            pltpu.VMEM((PER_W,), jnp.int32),
            pltpu.VMEM((NB, CH, ATOM), jnp.int32),
        ] + [pltpu.SemaphoreType.DMA] * (2 * NB),
    )
    def gk(tab_hbm, dst_hbm, out_hbm, idx_v, rows_v, *sems):
        gsems, ssems = sems[:NB], sems[NB:]
        wid = lax.axis_index("s") * 2 + lax.axis_index("c")
        base = wid * PER_W
        pltpu.sync_copy(dst_hbm.at[pl.ds(base, PER_W)], idx_v)

        def gdesc(c, bf):
            return pltpu.make_async_copy(
                tab_hbm.at[idx_v.at[pl.ds(c * CH, CH)]], rows_v.at[bf], gsems[bf])

        def sdesc(c, bf):
            return pltpu.make_async_copy(
                rows_v.at[bf], out_hbm.at[pl.ds(base + c * CH, CH)], ssems[bf])

        for c in range(STAG):
            gdesc(c, c % NB).start()

        def body(g, carry):
            for bf in range(NB):
                c = g * NB + bf

                @pl.when(c < N_CH)
                def _():
                    gdesc(c, bf).wait()
                    sdesc(c, bf).start()

                cn = c + STAG
                bfn = (bf + STAG) % NB

                @pl.when(cn < N_CH)
                def _():
                    @pl.when(cn >= NB)
                    def _():
                        sdesc(cn - NB, bfn).wait()

                    gdesc(cn, bfn).start()

            return carry

        lax.fori_loop(0, (N_CH + NB - 1) // NB, body, 0)
        for k in range(NB):
            c = N_CH - NB + k
            sdesc(c, c % NB).wait()

    return gk(table, dst)


def _edge_parts(md_r, ea_r, x_r, we_r, ws_r, bb_r):
    m_f, m_c = _unpack_halves(md_r[...])
    eab = ea_r[...].astype(jnp.bfloat16)
    ew = jnp.dot(eab, we_r[...], preferred_element_type=jnp.float32)
    a = jnp.dot(x_r[...], ws_r[...], preferred_element_type=jnp.float32)
    p_f = m_f + ew[:, :ATOM]
    p_c = m_c + ew[:, ATOM:]
    r = a + bb_r[...]
    return p_f, p_c, r


PH = NBLK


def _k_fused(md, ea, x, we, ws, bb, g1, b1, g2, b2):
    def body(md_r, ea_r, x_r, we_r, ws_r, bb_r, g1_r, b1_r, g2_r, b2_r,
             out_ref, sq_v, s2q2_v, s_v):
        i = pl.program_id(0)

        @pl.when(i == 0)
        def _():
            sq_v[...] = jnp.zeros_like(sq_v)
            s2q2_v[...] = jnp.zeros_like(s2q2_v)

        @pl.when(i < PH)
        def _():
            p_f, p_c, r = _edge_parts(md_r, ea_r, x_r, we_r, ws_r, bb_r)
            t_f = jnp.sum(p_f.reshape(G, NUM_NBR, ATOM), axis=1)
            t_c = jnp.sum(p_c.reshape(G, NUM_NBR, ATOM), axis=1)
            u_f = jnp.sum((p_f * p_f).reshape(G, NUM_NBR, ATOM), axis=1)
            u_c = jnp.sum((p_c * p_c).reshape(G, NUM_NBR, ATOM), axis=1)
            t = jnp.concatenate([t_f, t_c], axis=1)
            u = jnp.concatenate([u_f, u_c], axis=1)
            s_blk = jnp.sum(t + NUM_NBR * r, axis=0)
            q_blk = jnp.sum(u + 2.0 * (r * t) + NUM_NBR * (r * r), axis=0)
            sq_v[0:1, :] += s_blk[None, :]
            sq_v[1:2, :] += q_blk[None, :]

        @pl.when((i >= PH) & (i < 2 * PH))
        def _():
            mu = sq_v[0:1, :] * (1.0 / E)
            var = sq_v[1:2, :] * (1.0 / E) - mu * mu
            inv = lax.rsqrt(var + EPS)
            scale = g1_r[...] * inv
            shift = b1_r[...] - mu * scale

            p_f, p_c, r = _edge_parts(md_r, ea_r, x_r, we_r, ws_r, bb_r)
            shift_n = r * scale + shift
            hn_fh = (p_f * (0.5 * scale[:, :ATOM])).reshape(G, NUM_NBR, ATOM) \
                + (0.5 * shift_n[:, None, :ATOM])
            hn_c = (p_c * scale[:, ATOM:]).reshape(G, NUM_NBR, ATOM) \
                + shift_n[:, None, ATOM:]
            gate = (jnp.tanh(hn_fh) + 1.0) * _softplus(hn_c)
            ssum = 0.5 * jnp.sum(gate, axis=1)
            j = i - PH
            s_v[pl.ds(j * G, G), :] = ssum
            s2q2_v[0:1, :] += jnp.sum(ssum, axis=0)[None, :]
            s2q2_v[1:2, :] += jnp.sum(ssum * ssum, axis=0)[None, :]

        @pl.when(i >= 2 * PH)
        def _():
            mu2 = s2q2_v[0:1, :] * (1.0 / N)
            var2 = s2q2_v[1:2, :] * (1.0 / N) - mu2 * mu2
            inv2 = lax.rsqrt(var2 + EPS)
            j = i - 2 * PH
            ss = s_v[pl.ds(j * G, G), :]
            bn2 = (ss - mu2) * (inv2 * g2_r[...]) + b2_r[...]
            out_ref[...] = _softplus(x_r[...] + bn2)

    def _md_map(i):
        return (jnp.where(i < PH, i, jnp.minimum(i - PH, PH - 1)), 0)

    def _x_map(i):
        return (lax.rem(i, PH), 0)

    def _out_map(i):
        return (jnp.maximum(i - 2 * PH, 0), 0)

    return pl.pallas_call(
        body,
        grid=(3 * PH,),
        in_specs=[
            pl.BlockSpec((EB, ATOM), _md_map),
            pl.BlockSpec((EB, NBR_F), _md_map),
            pl.BlockSpec((G, ATOM), _x_map),
            pl.BlockSpec((NBR_F, F2), lambda i: (0, 0)),
            pl.BlockSpec((ATOM, F2), lambda i: (0, 0)),
            pl.BlockSpec((1, F2), lambda i: (0, 0)),
            pl.BlockSpec((1, F2), lambda i: (0, 0)),
            pl.BlockSpec((1, F2), lambda i: (0, 0)),
            pl.BlockSpec((1, ATOM), lambda i: (0, 0)),
            pl.BlockSpec((1, ATOM), lambda i: (0, 0)),
        ],
        out_specs=pl.BlockSpec((G, ATOM), _out_map),
        out_shape=jax.ShapeDtypeStruct((N, ATOM), jnp.float32),
        scratch_shapes=[
            pltpu.VMEM((2, F2), jnp.float32),
            pltpu.VMEM((2, ATOM), jnp.float32),
            pltpu.VMEM((N, ATOM), jnp.float32),
        ],
    )(md, ea, x, we, ws, bb, g1, b1, g2, b2)


def kernel(x, edge_index, edge_attr, W, b, g1, b1, g2, b2):
    dst = edge_index[1].astype(jnp.int32)
    Wt = W.T
    wd = Wt[:ATOM].astype(jnp.bfloat16)
    ws = Wt[ATOM:2 * ATOM]
    we = Wt[2 * ATOM:].astype(jnp.bfloat16)
    bb = b.reshape(1, F2)
    g1r = g1.reshape(1, F2)
    b1r = b1.reshape(1, F2)
    g2r = g2.reshape(1, ATOM)
    b2r = b2.reshape(1, ATOM)

    bmat = _k_pre(x, wd)
    md = _sc_gather(bmat, dst)
    return _k_fused(md, edge_attr, x, we, ws, bb, g1r, b1r, g2r, b2r)
